# baseline (speedup 1.0000x reference)
import functools

import jax
import jax.numpy as jnp
from jax import lax
from jax.experimental import pallas as pl
from jax.experimental.pallas import tpu as pltpu
from jax.experimental.pallas import tpu_sc as plsc

FEATS = 512
TEMP = 10000.0

NC = 2
NS = 16
L = 16
NW = NC * NS

N_ROWS = 16384
ROWS_PER_W = N_ROWS // NW
CHUNK = 16
N_CHUNKS = ROWS_PER_W // CHUNK
COLS = 4 * FEATS
NJ = COLS // L

_C0 = 0.9983365000243386
_C1 = -0.16221296264841442
_C2 = 0.0065211797336762294


def _freq_tables():
    f = jnp.arange(FEATS, dtype=jnp.float32)
    dim_t = TEMP ** (2.0 * jnp.floor(f / 2.0) / FEATS)
    w = (1.0 / dim_t).astype(jnp.float32)
    b = jnp.where((jnp.arange(FEATS) % 2) == 1, jnp.pi / 2, 0.0)
    return w, b.astype(jnp.float32)


def _sc_body(xrep_hbm, w_hbm, out_hbm,
             w_v, x0_v, x1_v, o0_v, o1_v, sem0, sem1, xsem0, xsem1):
    wid = lax.axis_index("s") * NC + lax.axis_index("c")
    row0 = wid * ROWS_PER_W

    pltpu.sync_copy(w_hbm, w_v)
    lane = lax.iota(jnp.int32, L)
    bvec = jnp.where((lane & 1) == 1, jnp.float32(jnp.pi / 2), jnp.float32(0.0))

    obufs = (o0_v, o1_v)
    sems = (sem0, sem1)
    xbufs = (x0_v, x1_v)
    xsems = (xsem0, xsem1)

    pltpu.async_copy(xrep_hbm.at[pl.ds(row0 * 4, CHUNK * 4)], x0_v, xsem0)

    def chunk_pair(i, _):
        for bsel in range(2):
            cc = i * 2 + bsel
            o_v = obufs[bsel]
            sem = sems[bsel]
            x_v = xbufs[bsel]
            r0 = row0 + cc * CHUNK
            pltpu.make_async_copy(
                xrep_hbm.at[pl.ds(0, CHUNK * 4)], x_v, xsems[bsel]
            ).wait()

            @pl.when(cc + 1 < N_CHUNKS)
            def _():
                pltpu.async_copy(
                    xrep_hbm.at[pl.ds((r0 + CHUNK) * 4, CHUNK * 4)],
                    xbufs[1 - bsel], xsems[1 - bsel])

            @pl.when(cc >= 2)
            def _():
                pltpu.make_async_copy(
                    o_v, out_hbm.at[pl.ds(r0, CHUNK)], sem
                ).wait()

            def row(r, _, o_v=o_v, x_v=x_v):
                for d in range(4):
                    x = x_v[r * 4 + d, :]

                    @plsc.parallel_loop(0, 32, unroll=16)
                    def col(j2, x=x, d=d, r=r, o_v=o_v):
                        jj = d * 32 + j2
                        w = w_v[jj, :]
                        t = x * w + bvec
                        u = t * t
                        p = u * _C2 + _C1
                        p = u * p + _C0
                        o_v[r, pl.ds(jj * L, L)] = t * p
                return 0

            lax.fori_loop(0, CHUNK, row, 0)
            pltpu.async_copy(o_v, out_hbm.at[pl.ds(r0, CHUNK)], sem)
        return 0

    lax.fori_loop(0, N_CHUNKS // 2, chunk_pair, 0)
    for bsel in range(2):
        pltpu.make_async_copy(
            obufs[bsel], out_hbm.at[pl.ds(row0, CHUNK)], sems[bsel]
        ).wait()


@jax.jit
def _run_sc(boxes):
    n = boxes.shape[0]
    w, _b = _freq_tables()
    w128 = jnp.tile(w, 4).reshape(NJ, L)
    xrep = jnp.broadcast_to(
        boxes.reshape(n * 4, 1), (n * 4, L)
    ).astype(jnp.float32)

    kern = pl.kernel(
        _sc_body,
        out_type=jax.ShapeDtypeStruct((n, COLS), jnp.float32),
        mesh=plsc.VectorSubcoreMesh(
            core_axis_name="c", subcore_axis_name="s",
            num_cores=NC, num_subcores=NS,
        ),
        scratch_types=[
            pltpu.VMEM((NJ, L), jnp.float32),
            pltpu.VMEM((CHUNK * 4, L), jnp.float32),
            pltpu.VMEM((CHUNK * 4, L), jnp.float32),
            pltpu.VMEM((CHUNK, COLS), jnp.float32),
            pltpu.VMEM((CHUNK, COLS), jnp.float32),
            pltpu.SemaphoreType.DMA,
            pltpu.SemaphoreType.DMA,
            pltpu.SemaphoreType.DMA,
            pltpu.SemaphoreType.DMA,
        ],
    )
    return kern(xrep, w128)


def kernel(boxes):
    if boxes.ndim == 3:
        boxes = boxes[0]
    return _run_sc(boxes)


# speedup vs baseline: 1.5245x; 1.5245x over previous
import functools

import jax
import jax.numpy as jnp
from jax import lax
from jax.experimental import pallas as pl
from jax.experimental.pallas import tpu as pltpu
from jax.experimental.pallas import tpu_sc as plsc

FEATS = 512
TEMP = 10000.0

NC = 2
NS = 16
L = 16
NW = NC * NS

N_ROWS = 16384
ROWS_PER_W = N_ROWS // NW
CHUNK = 16
N_CHUNKS = ROWS_PER_W // CHUNK
COLS = 4 * FEATS
NJ = COLS // L

_C0 = 0.9983365000243386
_C1 = -0.16221296264841442
_C2 = 0.0065211797336762294


def _freq_tables():
    f = jnp.arange(FEATS, dtype=jnp.float32)
    dim_t = TEMP ** (2.0 * jnp.floor(f / 2.0) / FEATS)
    w = (1.0 / dim_t).astype(jnp.float32)
    b = jnp.where((jnp.arange(FEATS) % 2) == 1, jnp.pi / 2, 0.0)
    return w, b.astype(jnp.float32)


def _sc_body(xrep_hbm, w_hbm, wm_hbm, out_hbm,
             w_v, wm_v, x0_v, x1_v, o0_v, o1_v, sem0, sem1, xsem0, xsem1):
    wid = lax.axis_index("s") * NC + lax.axis_index("c")
    row0 = wid * ROWS_PER_W

    pltpu.sync_copy(w_hbm, w_v)
    pltpu.sync_copy(wm_hbm, wm_v)
    lane = lax.iota(jnp.int32, L)
    odd = (lane & 1) == 1
    bvec = jnp.where(odd, jnp.float32(jnp.pi / 2), jnp.float32(0.0))
    cvec = jnp.where(odd, jnp.float32(1.0), jnp.float32(0.0))

    obufs = (o0_v, o1_v)
    sems = (sem0, sem1)
    xbufs = (x0_v, x1_v)
    xsems = (xsem0, xsem1)

    pltpu.async_copy(xrep_hbm.at[pl.ds(row0 * 4, CHUNK * 4)], x0_v, xsem0)

    def chunk_pair(i, _):
        for bsel in range(2):
            cc = i * 2 + bsel
            o_v = obufs[bsel]
            sem = sems[bsel]
            x_v = xbufs[bsel]
            r0 = row0 + cc * CHUNK
            pltpu.make_async_copy(
                xrep_hbm.at[pl.ds(0, CHUNK * 4)], x_v, xsems[bsel]
            ).wait()

            @pl.when(cc + 1 < N_CHUNKS)
            def _():
                pltpu.async_copy(
                    xrep_hbm.at[pl.ds((r0 + CHUNK) * 4, CHUNK * 4)],
                    xbufs[1 - bsel], xsems[1 - bsel])

            @pl.when(cc >= 2)
            def _():
                pltpu.make_async_copy(
                    o_v, out_hbm.at[pl.ds(r0, CHUNK)], sem
                ).wait()

            @plsc.parallel_loop(0, CHUNK // 2)
            def row(r2, o_v=o_v, x_v=x_v):
                r = r2 * 2
                for d in range(4):
                    x0 = x_v[r * 4 + d, :]
                    x1 = x_v[(r + 1) * 4 + d, :]

                    @plsc.parallel_loop(0, 8, unroll=8)
                    def col(j2, x0=x0, x1=x1, d=d, r=r, o_v=o_v):
                        jj = d * 32 + j2
                        w = w_v[jj, :]
                        for rr, x in ((r, x0), (r + 1, x1)):
                            t = x * w + bvec
                            u = t * t
                            p = u * _C2 + _C1
                            p = u * p + _C0
                            o_v[rr, pl.ds(jj * L, L)] = t * p

                    @plsc.parallel_loop(8, 32, unroll=8)
                    def lin(j2, x0=x0, x1=x1, d=d, r=r, o_v=o_v):
                        jj = d * 32 + j2
                        wm = wm_v[jj, :]
                        for rr, x in ((r, x0), (r + 1, x1)):
                            o_v[rr, pl.ds(jj * L, L)] = x * wm + cvec

            pltpu.async_copy(o_v, out_hbm.at[pl.ds(r0, CHUNK)], sem)
        return 0

    lax.fori_loop(0, N_CHUNKS // 2, chunk_pair, 0)
    for bsel in range(2):
        pltpu.make_async_copy(
            obufs[bsel], out_hbm.at[pl.ds(row0, CHUNK)], sems[bsel]
        ).wait()


@jax.jit
def _run_sc(boxes):
    n = boxes.shape[0]
    w, _b = _freq_tables()
    w128 = jnp.tile(w, 4).reshape(NJ, L)
    sin_lane = (jnp.arange(L) % 2) == 0
    wm128 = jnp.where(sin_lane[None, :], w128, 0.0).astype(jnp.float32)
    xrep = jnp.broadcast_to(
        boxes.reshape(n * 4, 1), (n * 4, L)
    ).astype(jnp.float32)

    kern = pl.kernel(
        _sc_body,
        out_type=jax.ShapeDtypeStruct((n, COLS), jnp.float32),
        mesh=plsc.VectorSubcoreMesh(
            core_axis_name="c", subcore_axis_name="s",
            num_cores=NC, num_subcores=NS,
        ),
        scratch_types=[
            pltpu.VMEM((NJ, L), jnp.float32),
            pltpu.VMEM((NJ, L), jnp.float32),
            pltpu.VMEM((CHUNK * 4, L), jnp.float32),
            pltpu.VMEM((CHUNK * 4, L), jnp.float32),
            pltpu.VMEM((CHUNK, COLS), jnp.float32),
            pltpu.VMEM((CHUNK, COLS), jnp.float32),
            pltpu.SemaphoreType.DMA,
            pltpu.SemaphoreType.DMA,
            pltpu.SemaphoreType.DMA,
            pltpu.SemaphoreType.DMA,
        ],
    )
    return kern(xrep, w128, wm128)


def kernel(boxes):
    if boxes.ndim == 3:
        boxes = boxes[0]
    return _run_sc(boxes)
